# baseline (speedup 1.0000x reference)
import functools

import jax
import jax.numpy as jnp
from jax.experimental import pallas as pl
from jax.experimental.pallas import tpu as pltpu

_RADIUS = 0.1
_QT = 128
_NTS = 1024
_CH = 128
_LT = 128


def _lift_kernel(pnd_ref, wt_ref, b_ref, f_ref):
    x = pnd_ref[...]
    b_dim, n2, in2 = x.shape
    y = jnp.dot(x.reshape(b_dim * n2, in2), wt_ref[...],
                preferred_element_type=jnp.float32) + b_ref[...]
    f_ref[...] = y.reshape(b_dim, n2, -1)


def _enc_kernel(q_ref, xt_ref, xs6_ref, xb_ref, f2_ref, w1q2_ref, b12_ref,
                w1x6_ref, w22_ref, b22_ref, w32_ref, b32_ref,
                out_ref, acc_ref, cnt_ref, *, r2, batch, out_ch):
    j = pl.program_id(1)
    nj = pl.num_programs(1)
    q = q_ref[...]
    xt = xt_ref[...]
    d2 = None
    for c in range(3):
        dc = q[:, c:c + 1] - xt[c:c + 1, :]
        s = dc * dc
        d2 = s if d2 is None else d2 + s
    mask = d2 <= r2

    @pl.when(j == 0)
    def _init():
        acc_ref[...] = jnp.zeros_like(acc_ref)
        cnt_ref[...] = jnp.zeros_like(cnt_ref)

    qw2 = jnp.dot(q, w1q2_ref[...],
                  preferred_element_type=jnp.float32) + b12_ref[...]
    qt = q.shape[0]
    lanes = 2 * out_ch
    n_chunks = _NTS // _CH
    p_ch = _CH // 2

    for ch in range(n_chunks):
        cm = mask[:, _CH * ch:_CH * (ch + 1)]
        p0 = p_ch * ch

        @pl.when(jnp.any(cm))
        def _chunk(cm=cm, p0=p0):
            cnt_ref[...] += jnp.sum(cm.astype(jnp.float32), axis=1,
                                    keepdims=True)
            xs6 = xs6_ref[p0:p0 + p_ch, :]
            xw2 = jnp.dot(xs6, w1x6_ref[...],
                          preferred_element_type=jnp.float32)
            h1 = jax.nn.gelu(qw2[:, None, :] + xw2[None, :, :])
            h2 = jax.nn.gelu(
                jnp.dot(h1.reshape(qt * p_ch, lanes), w22_ref[...],
                        preferred_element_type=jnp.float32) + b22_ref[...])
            k2 = jnp.dot(h2, w32_ref[...],
                         preferred_element_type=jnp.float32) + b32_ref[...]
            k2 = k2.reshape(qt, p_ch, lanes)
            xb = xb_ref[:, p0:p0 + p_ch, :]
            d2p = None
            for c in range(3):
                qb = jnp.broadcast_to(q[:, c:c + 1], (qt, lanes))
                dc = qb[:, None, :] - xb[c][None, :, :]
                s = dc * dc
                d2p = s if d2p is None else d2p + s
            k2 = k2 * (d2p <= r2).astype(jnp.float32)
            for b in range(batch):
                acc_ref[b] += jnp.sum(
                    k2 * f2_ref[b, p0:p0 + p_ch, :][None, :, :], axis=1)

    @pl.when(j == nj - 1)
    def _fini():
        denom = jnp.maximum(cnt_ref[...], 1.0)
        acc = acc_ref[...]
        out_ref[...] = ((acc[:, :, 0:out_ch] + acc[:, :, out_ch:])
                        / denom[None, :, :])


def kernel(x_coord, pndata, latent_tokens_coord, W_lift, b_lift,
           W1, b1, W2, b2, W3, b3):
    num_nodes = x_coord.shape[0]
    num_latent = latent_tokens_coord.shape[0]
    batch, _, in_ch = pndata.shape
    out_ch = W_lift.shape[0]
    hid = W1.shape[1]
    r2 = _RADIUS * _RADIUS
    n2 = num_nodes // 2
    nt2s = _NTS // 2

    def cell_code(c):
        g = jnp.clip(jnp.floor(c * (1.0 / _RADIUS)), 0, 9).astype(jnp.int32)
        return (g[:, 0] * 10 + g[:, 1]) * 10 + g[:, 2]

    perm_n = jnp.argsort(cell_code(x_coord))
    perm_q = jnp.argsort(cell_code(latent_tokens_coord))
    xs = x_coord[perm_n]
    qs = latent_tokens_coord[perm_q]
    pnds = pndata[:, perm_n, :]

    xt8 = jnp.zeros((8, num_nodes), jnp.float32).at[0:3, :].set(xs.T)
    xs6 = xs.reshape(n2, 6)
    xb = jnp.concatenate([
        jnp.broadcast_to(xs[0::2].T[:, :, None], (3, n2, out_ch)),
        jnp.broadcast_to(xs[1::2].T[:, :, None], (3, n2, out_ch)),
    ], axis=2)

    def blockdiag(w):
        r, c = w.shape
        z = jnp.zeros((2 * r, 2 * c), w.dtype)
        return z.at[0:r, 0:c].set(w).at[r:, c:].set(w)

    wl2 = blockdiag(W_lift.T)
    bl2 = jnp.tile(b_lift.reshape(1, -1), (1, 2))
    w1x6 = blockdiag(W1[0:3, :])
    w1q2 = jnp.tile(W1[3:6, :], (1, 2))
    b12 = jnp.tile(b1.reshape(1, -1), (1, 2))
    w22 = blockdiag(W2)
    b22 = jnp.tile(b2.reshape(1, -1), (1, 2))
    w32 = blockdiag(W3)
    b32 = jnp.tile(b3.reshape(1, -1), (1, 2))

    pnd6 = pnds.reshape(batch, n2, 2 * in_ch)
    lift_tiles = num_nodes // _LT
    lt2 = _LT // 2
    f2 = pl.pallas_call(
        _lift_kernel,
        grid=(lift_tiles,),
        in_specs=[
            pl.BlockSpec((batch, lt2, 2 * in_ch), lambda i: (0, i, 0)),
            pl.BlockSpec((2 * in_ch, 2 * out_ch), lambda i: (0, 0)),
            pl.BlockSpec((1, 2 * out_ch), lambda i: (0, 0)),
        ],
        out_specs=pl.BlockSpec((batch, lt2, 2 * out_ch), lambda i: (0, i, 0)),
        out_shape=jax.ShapeDtypeStruct((batch, n2, 2 * out_ch), jnp.float32),
    )(pnd6, wl2, bl2)

    q_tiles = num_latent // _QT
    j_tiles = num_nodes // _NTS
    enc = functools.partial(_enc_kernel, r2=r2, batch=batch, out_ch=out_ch)
    out_sorted = pl.pallas_call(
        enc,
        grid=(q_tiles, j_tiles),
        in_specs=[
            pl.BlockSpec((_QT, 3), lambda i, j: (i, 0)),
            pl.BlockSpec((8, _NTS), lambda i, j: (0, j)),
            pl.BlockSpec((nt2s, 6), lambda i, j: (j, 0)),
            pl.BlockSpec((3, nt2s, 2 * out_ch), lambda i, j: (0, j, 0)),
            pl.BlockSpec((batch, nt2s, 2 * out_ch), lambda i, j: (0, j, 0)),
            pl.BlockSpec((3, 2 * hid), lambda i, j: (0, 0)),
            pl.BlockSpec((1, 2 * hid), lambda i, j: (0, 0)),
            pl.BlockSpec((6, 2 * hid), lambda i, j: (0, 0)),
            pl.BlockSpec((2 * hid, 2 * hid), lambda i, j: (0, 0)),
            pl.BlockSpec((1, 2 * hid), lambda i, j: (0, 0)),
            pl.BlockSpec((2 * hid, 2 * out_ch), lambda i, j: (0, 0)),
            pl.BlockSpec((1, 2 * out_ch), lambda i, j: (0, 0)),
        ],
        out_specs=pl.BlockSpec((batch, _QT, out_ch), lambda i, j: (0, i, 0)),
        out_shape=jax.ShapeDtypeStruct((batch, num_latent, out_ch),
                                       jnp.float32),
        scratch_shapes=[
            pltpu.VMEM((batch, _QT, 2 * out_ch), jnp.float32),
            pltpu.VMEM((_QT, 1), jnp.float32),
        ],
    )(qs, xt8, xs6, xb, f2, w1q2, b12, w1x6, w22, b22, w32, b32)

    inv_q = jnp.argsort(perm_q)
    return out_sorted[:, inv_q, :]


# speedup vs baseline: 4.2794x; 4.2794x over previous
import functools

import jax
import jax.numpy as jnp
from jax.experimental import pallas as pl
from jax.experimental.pallas import tpu as pltpu

_RADIUS = 0.1
_QT = 64
_NTS = 1024
_CH = 128
_LT = 128


def _lift_kernel(pnd_ref, wt_ref, b_ref, f_ref):
    x = pnd_ref[...]
    b_dim, n2, in2 = x.shape
    y = jnp.dot(x.reshape(b_dim * n2, in2), wt_ref[...],
                preferred_element_type=jnp.float32) + b_ref[...]
    f_ref[...] = y.reshape(b_dim, n2, -1)


def _enc_kernel(q_ref, xt_ref, xs6_ref, xb_ref, f2_ref, w1q2_ref, b12_ref,
                w1x6_ref, w22_ref, b22_ref, w32_ref, b32_ref,
                out_ref, acc_ref, cnt_ref, *, r2, batch, out_ch):
    j = pl.program_id(1)
    nj = pl.num_programs(1)
    q = q_ref[...]

    @pl.when(j == 0)
    def _init():
        acc_ref[...] = jnp.zeros_like(acc_ref)
        cnt_ref[...] = jnp.zeros_like(cnt_ref)

    qw2 = jnp.dot(q, w1q2_ref[...],
                  preferred_element_type=jnp.float32) + b12_ref[...]
    qt = q.shape[0]
    lanes = 2 * out_ch
    n_chunks = _NTS // _CH
    p_ch = _CH // 2

    for ch in range(n_chunks):
        xtc = xt_ref[0:8, _CH * ch:_CH * (ch + 1)]
        d2 = None
        for c in range(3):
            dc = q[:, c:c + 1] - xtc[c:c + 1, :]
            s = dc * dc
            d2 = s if d2 is None else d2 + s
        cm = d2 <= r2
        p0 = p_ch * ch

        @pl.when(jnp.any(cm))
        def _chunk(cm=cm, p0=p0):
            cnt_ref[...] += jnp.sum(cm.astype(jnp.float32), axis=1,
                                    keepdims=True)
            xs6 = xs6_ref[p0:p0 + p_ch, :]
            xw2 = jnp.dot(xs6, w1x6_ref[...],
                          preferred_element_type=jnp.float32)
            h1 = jax.nn.gelu(qw2[:, None, :] + xw2[None, :, :])
            h2 = jax.nn.gelu(
                jnp.dot(h1.reshape(qt * p_ch, lanes), w22_ref[...],
                        preferred_element_type=jnp.float32) + b22_ref[...])
            k2 = jnp.dot(h2, w32_ref[...],
                         preferred_element_type=jnp.float32) + b32_ref[...]
            k2 = k2.reshape(qt, p_ch, lanes)
            xb = xb_ref[:, p0:p0 + p_ch, :]
            d2p = None
            for c in range(3):
                qb = jnp.broadcast_to(q[:, c:c + 1], (qt, lanes))
                dc = qb[:, None, :] - xb[c][None, :, :]
                s = dc * dc
                d2p = s if d2p is None else d2p + s
            k2 = k2 * (d2p <= r2).astype(jnp.float32)
            for b in range(batch):
                acc_ref[b] += jnp.sum(
                    k2 * f2_ref[b, p0:p0 + p_ch, :][None, :, :], axis=1)

    @pl.when(j == nj - 1)
    def _fini():
        denom = jnp.maximum(cnt_ref[...], 1.0)
        acc = acc_ref[...]
        out_ref[...] = ((acc[:, :, 0:out_ch] + acc[:, :, out_ch:])
                        / denom[None, :, :])


def kernel(x_coord, pndata, latent_tokens_coord, W_lift, b_lift,
           W1, b1, W2, b2, W3, b3):
    num_nodes = x_coord.shape[0]
    num_latent = latent_tokens_coord.shape[0]
    batch, _, in_ch = pndata.shape
    out_ch = W_lift.shape[0]
    hid = W1.shape[1]
    r2 = _RADIUS * _RADIUS
    n2 = num_nodes // 2
    nt2s = _NTS // 2

    def cell_code(c):
        g = jnp.clip(jnp.floor(c * (1.0 / _RADIUS)), 0, 9).astype(jnp.int32)
        return (g[:, 0] * 10 + g[:, 1]) * 10 + g[:, 2]

    perm_n = jnp.argsort(cell_code(x_coord))
    perm_q = jnp.argsort(cell_code(latent_tokens_coord))
    xs = x_coord[perm_n]
    qs = latent_tokens_coord[perm_q]
    pnds = pndata[:, perm_n, :]

    xt8 = jnp.zeros((8, num_nodes), jnp.float32).at[0:3, :].set(xs.T)
    xs6 = xs.reshape(n2, 6)
    xb = jnp.concatenate([
        jnp.broadcast_to(xs[0::2].T[:, :, None], (3, n2, out_ch)),
        jnp.broadcast_to(xs[1::2].T[:, :, None], (3, n2, out_ch)),
    ], axis=2)

    def blockdiag(w):
        r, c = w.shape
        z = jnp.zeros((2 * r, 2 * c), w.dtype)
        return z.at[0:r, 0:c].set(w).at[r:, c:].set(w)

    wl2 = blockdiag(W_lift.T)
    bl2 = jnp.tile(b_lift.reshape(1, -1), (1, 2))
    w1x6 = blockdiag(W1[0:3, :])
    w1q2 = jnp.tile(W1[3:6, :], (1, 2))
    b12 = jnp.tile(b1.reshape(1, -1), (1, 2))
    w22 = blockdiag(W2)
    b22 = jnp.tile(b2.reshape(1, -1), (1, 2))
    w32 = blockdiag(W3)
    b32 = jnp.tile(b3.reshape(1, -1), (1, 2))

    pnd6 = pnds.reshape(batch, n2, 2 * in_ch)
    lift_tiles = num_nodes // _LT
    lt2 = _LT // 2
    f2 = pl.pallas_call(
        _lift_kernel,
        grid=(lift_tiles,),
        in_specs=[
            pl.BlockSpec((batch, lt2, 2 * in_ch), lambda i: (0, i, 0)),
            pl.BlockSpec((2 * in_ch, 2 * out_ch), lambda i: (0, 0)),
            pl.BlockSpec((1, 2 * out_ch), lambda i: (0, 0)),
        ],
        out_specs=pl.BlockSpec((batch, lt2, 2 * out_ch), lambda i: (0, i, 0)),
        out_shape=jax.ShapeDtypeStruct((batch, n2, 2 * out_ch), jnp.float32),
    )(pnd6, wl2, bl2)

    q_tiles = num_latent // _QT
    j_tiles = num_nodes // _NTS
    enc = functools.partial(_enc_kernel, r2=r2, batch=batch, out_ch=out_ch)
    out_sorted = pl.pallas_call(
        enc,
        grid=(q_tiles, j_tiles),
        in_specs=[
            pl.BlockSpec((_QT, 3), lambda i, j: (i, 0)),
            pl.BlockSpec((8, _NTS), lambda i, j: (0, j)),
            pl.BlockSpec((nt2s, 6), lambda i, j: (j, 0)),
            pl.BlockSpec((3, nt2s, 2 * out_ch), lambda i, j: (0, j, 0)),
            pl.BlockSpec((batch, nt2s, 2 * out_ch), lambda i, j: (0, j, 0)),
            pl.BlockSpec((3, 2 * hid), lambda i, j: (0, 0)),
            pl.BlockSpec((1, 2 * hid), lambda i, j: (0, 0)),
            pl.BlockSpec((6, 2 * hid), lambda i, j: (0, 0)),
            pl.BlockSpec((2 * hid, 2 * hid), lambda i, j: (0, 0)),
            pl.BlockSpec((1, 2 * hid), lambda i, j: (0, 0)),
            pl.BlockSpec((2 * hid, 2 * out_ch), lambda i, j: (0, 0)),
            pl.BlockSpec((1, 2 * out_ch), lambda i, j: (0, 0)),
        ],
        out_specs=pl.BlockSpec((batch, _QT, out_ch), lambda i, j: (0, i, 0)),
        out_shape=jax.ShapeDtypeStruct((batch, num_latent, out_ch),
                                       jnp.float32),
        scratch_shapes=[
            pltpu.VMEM((batch, _QT, 2 * out_ch), jnp.float32),
            pltpu.VMEM((_QT, 1), jnp.float32),
        ],
    )(qs, xt8, xs6, xb, f2, w1q2, b12, w1x6, w22, b22, w32, b32)

    inv_q = jnp.argsort(perm_q)
    return out_sorted[:, inv_q, :]
